# baseline (speedup 1.0000x reference)
import functools

import numpy as np
import jax
import jax.numpy as jnp
from jax.experimental import pallas as pl

_PROB = 0.25


@functools.lru_cache(maxsize=None)
def _dropped_ids(b):
    num = 1 if b == 1 else int(b * _PROB)
    with jax.ensure_compile_time_eval():
        perm = np.asarray(jax.random.permutation(jax.random.key(1), b))
    return tuple(int(x) for x in perm[:num])


def kernel(emb0):
    b, t, c, d = emb0.shape
    dropped = _dropped_ids(b)
    x = emb0.reshape(b, t, c * d)
    nt = 8
    tb = t // nt

    def body(in_ref, out_ref):
        bi = pl.program_id(0)
        ti = pl.program_id(1)
        is_drop = functools.reduce(
            jnp.logical_or, [bi == i for i in dropped], jnp.bool_(False)
        )
        tg = ti * tb + jax.lax.broadcasted_iota(jnp.int32, (1, tb, 1), 1)
        mask = jnp.logical_and(is_drop, tg < t - 1)
        out_ref[...] = jnp.where(mask, 0.0, in_ref[...])

    out = pl.pallas_call(
        body,
        grid=(b, nt),
        in_specs=[pl.BlockSpec((1, tb, c * d), lambda i, j: (i, j, 0))],
        out_specs=pl.BlockSpec((1, tb, c * d), lambda i, j: (i, j, 0)),
        out_shape=jax.ShapeDtypeStruct((b, t, c * d), emb0.dtype),
    )(x)
    return out.reshape(b, t, c, d)


# speedup vs baseline: 1.2953x; 1.2953x over previous
import functools

import numpy as np
import jax
import jax.numpy as jnp
from jax.experimental import pallas as pl

_PROB = 0.25


@functools.lru_cache(maxsize=None)
def _dropped_ids(b):
    num = 1 if b == 1 else int(b * _PROB)
    with jax.ensure_compile_time_eval():
        perm = np.asarray(jax.random.permutation(jax.random.key(1), b))
    return tuple(int(x) for x in perm[:num])


def kernel(emb0):
    b, t, c, d = emb0.shape
    cd = c * d
    dropped = set(_dropped_ids(b))

    redir = list(range(b))
    kept_before = None
    for i in range(b):
        if i in dropped:
            redir[i] = kept_before if kept_before is not None else -1
        else:
            kept_before = i
    first_kept = next(i for i in range(b) if i not in dropped)
    redir = [first_kept if r == -1 else r for r in redir]

    def _redirect(i):
        r = i
        for db in sorted(dropped):
            r = jnp.where(i == db, redir[db], r)
        return r

    x = emb0.reshape(b, t, cd)

    def body(in_ref, last_ref, out_ref):
        bi = pl.program_id(0)
        is_drop = functools.reduce(
            jnp.logical_or, [bi == i for i in dropped], jnp.bool_(False)
        )

        @pl.when(jnp.logical_not(is_drop))
        def _copy():
            out_ref[...] = in_ref[...]

        @pl.when(is_drop)
        def _zero():
            out_ref[...] = jnp.zeros(out_ref.shape, out_ref.dtype)
            out_ref[0, t - 1, :] = last_ref[0, 7, :]

    out = pl.pallas_call(
        body,
        grid=(b,),
        in_specs=[
            pl.BlockSpec((1, t, cd), lambda i: (_redirect(i), 0, 0)),
            pl.BlockSpec((1, 8, cd), lambda i: (i, t // 8 - 1, 0)),
        ],
        out_specs=pl.BlockSpec((1, t, cd), lambda i: (i, 0, 0)),
        out_shape=jax.ShapeDtypeStruct((b, t, cd), emb0.dtype),
    )(x, x)
    return out.reshape(b, t, c, d)


# speedup vs baseline: 6.1595x; 4.7553x over previous
import functools

import numpy as np
import jax
import jax.numpy as jnp
from jax.experimental import pallas as pl

_PROB = 0.25


@functools.lru_cache(maxsize=None)
def _dropped_ids(b):
    num = 1 if b == 1 else int(b * _PROB)
    with jax.ensure_compile_time_eval():
        perm = np.asarray(jax.random.permutation(jax.random.key(1), b))
    return tuple(int(x) for x in perm[:num])


def kernel(emb0):
    b, t, c, d = emb0.shape
    dropped = set(_dropped_ids(b))

    redir = list(range(b))
    kept_before = None
    for i in range(b):
        if i in dropped:
            redir[i] = kept_before if kept_before is not None else -1
        else:
            kept_before = i
    first_kept = next(i for i in range(b) if i not in dropped)
    redir = [first_kept if r == -1 else r for r in redir]

    def _redirect(i):
        r = i
        for db in sorted(dropped):
            r = jnp.where(i == db, redir[db], r)
        return r

    def body(in_ref, last_ref, out_ref):
        bi = pl.program_id(0)
        is_drop = functools.reduce(
            jnp.logical_or, [bi == i for i in dropped], jnp.bool_(False)
        )

        @pl.when(jnp.logical_not(is_drop))
        def _copy():
            out_ref[...] = in_ref[...]

        @pl.when(is_drop)
        def _zero():
            out_ref[...] = jnp.zeros(out_ref.shape, out_ref.dtype)
            out_ref[0, t - 1, :, :] = last_ref[0, 0, :, :]

    out = pl.pallas_call(
        body,
        grid=(b,),
        in_specs=[
            pl.BlockSpec((1, t, c, d), lambda i: (_redirect(i), 0, 0, 0)),
            pl.BlockSpec((1, 1, c, d), lambda i: (i, t - 1, 0, 0)),
        ],
        out_specs=pl.BlockSpec((1, t, c, d), lambda i: (i, 0, 0, 0)),
        out_shape=jax.ShapeDtypeStruct((b, t, c, d), emb0.dtype),
    )(emb0, emb0)
    return out
